# Initial kernel scaffold; baseline (speedup 1.0000x reference)
import jax
import jax.numpy as jnp
from jax.experimental import pallas as pl


def kernel(x, edge_index, batch, c1W, c1b, c2W, c2b, c3W, c3b, c4W, c4b, bn1_g, bn1_b, bn3_g, bn3_b, a3, fc1W, fc1b, fc2W, fc2b, fc3W, fc3b):
    raise NotImplementedError("write your pallas kernel here")


# speedup vs baseline: 3.7541x; 3.7541x over previous
import functools

import jax
import jax.numpy as jnp
from jax.experimental import pallas as pl
from jax.experimental.pallas import tpu as pltpu

_N, _D, _H, _C, _G = 10000, 128, 128, 10, 64
_BN = 2000
_EPS = 1e-5


def _fused_kernel(x_ref, batch_ref, w1_ref, b1_ref, w2_ref, b2_ref, w3_ref,
                  b3_ref, w4_ref, b4_ref, g4_ref, c4_ref, f1w_ref, f1b_ref,
                  f2w_ref, f2b_ref, f3w_ref, f3b_ref, a3_ref, out_ref,
                  acc_ref):
    i = pl.program_id(0)

    h = jnp.maximum(jnp.dot(x_ref[...], w1_ref[...],
                            preferred_element_type=jnp.float32) + b1_ref[...],
                    0.0)
    h = jnp.maximum(jnp.dot(h, w2_ref[...],
                            preferred_element_type=jnp.float32) + b2_ref[...],
                    0.0)
    h = jnp.maximum(jnp.dot(h, w3_ref[...],
                            preferred_element_type=jnp.float32) + b3_ref[...],
                    0.0)
    h = jnp.maximum(jnp.dot(h, w4_ref[...],
                            preferred_element_type=jnp.float32) + b4_ref[...],
                    0.0)
    h = h * g4_ref[...] + c4_ref[...]

    ids = batch_ref[0, 0, :]
    seg = jax.lax.broadcasted_iota(jnp.int32, (_BN, _G), 1)
    oh = (ids[:, None] == seg).astype(jnp.float32)
    part = jax.lax.dot_general(oh, h, (((0,), (0,)), ((), ())),
                               preferred_element_type=jnp.float32)

    @pl.when(i == 0)
    def _init():
        acc_ref[...] = part

    @pl.when(i > 0)
    def _accum():
        acc_ref[...] = acc_ref[...] + part

    @pl.when(i == pl.num_programs(0) - 1)
    def _head():
        s = acc_ref[...]
        a3 = a3_ref[0, 0]
        t = jnp.dot(s, f1w_ref[...],
                    preferred_element_type=jnp.float32) + f1b_ref[...]
        t = jnp.where(t >= 0, t, a3 * t)
        t = jnp.dot(t, f2w_ref[...],
                    preferred_element_type=jnp.float32) + f2b_ref[...]
        t = jax.nn.sigmoid(t)
        out_ref[...] = jnp.dot(t, f3w_ref[...],
                               preferred_element_type=jnp.float32) + f3b_ref[...]


@jax.jit
def kernel(x, edge_index, batch, c1W, c1b, c2W, c2b, c3W, c3b, c4W, c4b,
           bn1_g, bn1_b, bn3_g, bn3_b, a3, fc1W, fc1b, fc2W, fc2b, fc3W,
           fc3b):
    del edge_index
    inv_s = 1.0 / jnp.sqrt(1.0 + _EPS)

    w1 = (c1W * (bn1_g * inv_s)[None, :]).T
    b1 = (bn1_b @ c1W.T + c1b)[None, :]
    w2 = c2W.T
    b2 = c2b[None, :]
    w3 = (c3W * (bn3_g * inv_s)[None, :]).T
    b3 = (bn3_b @ c3W.T + c3b)[None, :]
    w4 = (c4W * (bn3_g * inv_s)[None, :]).T
    b4 = (bn3_b @ c4W.T + c4b)[None, :]
    g4 = (bn3_g * inv_s)[None, :]
    c4 = bn3_b[None, :]

    f1w = fc1W.T
    f1b = fc1b[None, :]
    f2w = fc2W.T
    f2b = fc2b[None, :]
    f3w = fc3W.T
    f3b = fc3b[None, :]
    a3m = a3.reshape(1, 1)

    batch3d = batch.reshape(_N // _BN, 1, _BN)

    grid = _N // _BN
    full = lambda i: (0, 0)
    out = pl.pallas_call(
        _fused_kernel,
        grid=(grid,),
        in_specs=[
            pl.BlockSpec((_BN, _D), lambda i: (i, 0)),
            pl.BlockSpec((1, 1, _BN), lambda i: (i, 0, 0)),
            pl.BlockSpec((_D, _H), full),
            pl.BlockSpec((1, _H), full),
            pl.BlockSpec((_H, _H), full),
            pl.BlockSpec((1, _H), full),
            pl.BlockSpec((_H, _H), full),
            pl.BlockSpec((1, _H), full),
            pl.BlockSpec((_H, _H), full),
            pl.BlockSpec((1, _H), full),
            pl.BlockSpec((1, _H), full),
            pl.BlockSpec((1, _H), full),
            pl.BlockSpec((_H, _H), full),
            pl.BlockSpec((1, _H), full),
            pl.BlockSpec((_H, _H // 2), full),
            pl.BlockSpec((1, _H // 2), full),
            pl.BlockSpec((_H // 2, _C), full),
            pl.BlockSpec((1, _C), full),
            pl.BlockSpec((1, 1), full),
        ],
        out_specs=pl.BlockSpec((_G, _C), full),
        out_shape=jax.ShapeDtypeStruct((_G, _C), jnp.float32),
        scratch_shapes=[pltpu.VMEM((_G, _H), jnp.float32)],
    )(x, batch3d, w1, b1, w2, b2, w3, b3, w4, b4, g4, c4, f1w, f1b, f2w,
      f2b, f3w, f3b, a3m)
    return out
